# baseline (speedup 1.0000x reference)
import functools

import jax
import jax.numpy as jnp
from jax import lax
from jax.experimental import pallas as pl
from jax.experimental.pallas import tpu as pltpu
from jax.experimental.pallas import tpu_sc as plsc

N = 10000
E = 160000
D = 256
HD = 128

NC = 2
NS = 16
CHUNK = 128

EPAD = 163840
PAD_DST = 10008

ROWS = 10240

TCH = EPAD // NS
NCH = TCH // CHUNK

_mesh = plsc.VectorSubcoreMesh(core_axis_name="c", subcore_axis_name="s")


@functools.partial(
    pl.kernel,
    mesh=_mesh,
    out_type=jax.ShapeDtypeStruct((NC, ROWS), jnp.float32),
    scratch_types=[
        pltpu.VMEM((CHUNK,), jnp.int32),
        pltpu.VMEM((CHUNK,), jnp.float32),
        pltpu.VMEM((640,), jnp.float32),
        pltpu.VMEM_SHARED((ROWS,), jnp.float32),
    ],
)
def _sc_degree(dst_hbm, ones_hbm, zeros_hbm, out_hbm, idx_v, ones_v, z_v,
               acc_sh):
    c = lax.axis_index("c")
    s = lax.axis_index("s")

    pltpu.sync_copy(zeros_hbm, z_v)
    pltpu.sync_copy(z_v, acc_sh.at[pl.ds(s * 640, 640)])
    pltpu.sync_copy(ones_hbm, ones_v)
    plsc.subcore_barrier()

    base = (c * NS + s) * (EPAD // (NC * NS))

    @pl.loop(0, EPAD // (NC * NS), step=CHUNK)
    def _(k):
        pltpu.sync_copy(dst_hbm.at[pl.ds(base + k, CHUNK)], idx_v)
        pltpu.sync_copy(ones_v, acc_sh.at[idx_v], add=True)

    plsc.subcore_barrier()
    pltpu.sync_copy(acc_sh.at[pl.ds(s * 640, 640)], z_v)
    pltpu.sync_copy(z_v, out_hbm.at[c, pl.ds(s * 640, 640)])


@functools.partial(
    pl.kernel,
    mesh=_mesh,
    out_type=jax.ShapeDtypeStruct((NC, ROWS, HD), jnp.float32),
    scratch_types=[
        pltpu.VMEM((TCH,), jnp.int32),
        pltpu.VMEM((CHUNK,), jnp.int32),
        pltpu.VMEM((CHUNK,), jnp.int32),
        pltpu.VMEM((CHUNK, HD), jnp.float32),
        pltpu.VMEM((CHUNK, HD), jnp.float32),
        pltpu.VMEM_SHARED((ROWS, HD), jnp.float32),
        pltpu.SemaphoreType.DMA,
        pltpu.SemaphoreType.DMA,
        pltpu.SemaphoreType.DMA,
        pltpu.SemaphoreType.DMA,
    ],
)
def _sc_segsum(src_hbm, dst_hbm, h3_hbm, zeros_hbm, out_hbm,
               src_all, d0, d1, r0, r1, acc_sh, sem0, sem1, ssem0, ssem1):
    c = lax.axis_index("c")
    s = lax.axis_index("s")
    slots = ((d0, r0, sem0, ssem0), (d1, r1, sem1, ssem1))
    ebase = s * TCH
    h2c = h3_hbm.at[c]

    def fire(k, slot, first=False):
        dv, rv, sem, ssem = slots[slot]
        if not first:
            pltpu.make_async_copy(rv, acc_sh.at[dv], ssem).wait()
        pltpu.async_copy(h2c.at[src_all.at[pl.ds(k * CHUNK, CHUNK)]], rv, sem)
        pltpu.async_copy(dst_hbm.at[pl.ds(ebase + k * CHUNK, CHUNK)], dv, sem)

    def drain(k, slot):
        dv, rv, sem, ssem = slots[slot]
        pltpu.make_async_copy(
            h2c.at[src_all.at[pl.ds(k * CHUNK, CHUNK)]], rv, sem).wait()
        pltpu.make_async_copy(
            dst_hbm.at[pl.ds(ebase + k * CHUNK, CHUNK)], dv, sem).wait()
        pltpu.async_copy(rv, acc_sh.at[dv], ssem, add=True)

    pltpu.sync_copy(src_hbm.at[pl.ds(ebase, TCH)], src_all)

    zbase = s * 640
    for t in range(5):
        pltpu.sync_copy(zeros_hbm, acc_sh.at[pl.ds(zbase + t * CHUNK, CHUNK)])

    fire(0, 0, first=True)
    fire(1, 1, first=True)
    plsc.subcore_barrier()

    @pl.loop(0, NCH, step=2)
    def _(k):
        drain(k, 0)

        @pl.when(k + 2 < NCH)
        def _():
            fire(k + 2, 0)

        drain(k + 1, 1)

        @pl.when(k + 3 < NCH)
        def _():
            fire(k + 3, 1)

    pltpu.make_async_copy(r0, acc_sh.at[d0], ssem0).wait()
    pltpu.make_async_copy(r1, acc_sh.at[d1], ssem1).wait()
    plsc.subcore_barrier()
    pltpu.sync_copy(acc_sh.at[pl.ds(zbase, 640)],
                    out_hbm.at[c, pl.ds(zbase, 640)])


def _tc_matmul(x, W):
    B = 1000

    def body(x_ref, w_ref, o_ref):
        h = jnp.dot(x_ref[...], w_ref[...],
                    preferred_element_type=jnp.float32,
                    precision=lax.Precision.HIGHEST)
        o_ref[0] = h[:, :HD]
        o_ref[1] = h[:, HD:]

    return pl.pallas_call(
        body,
        grid=(N // B,),
        in_specs=[
            pl.BlockSpec((B, D), lambda i: (i, 0)),
            pl.BlockSpec((D, D), lambda i: (0, 0)),
        ],
        out_specs=pl.BlockSpec((NC, B, HD), lambda i: (0, i, 0)),
        out_shape=jax.ShapeDtypeStruct((NC, N, HD), jnp.float32),
    )(x, W)


def _tc_scale(h3raw, deg_parts):
    B = 1000

    def body(h_ref, dp_ref, o_ref):
        d = dp_ref[0] + dp_ref[1] + 1.0
        dinv = lax.rsqrt(d)
        o_ref[...] = h_ref[...] * dinv

    return pl.pallas_call(
        body,
        grid=(N // B,),
        in_specs=[
            pl.BlockSpec((NC, B, HD), lambda i: (0, i, 0)),
            pl.BlockSpec((NC, B, 1), lambda i: (0, i, 0)),
        ],
        out_specs=pl.BlockSpec((NC, B, HD), lambda i: (0, i, 0)),
        out_shape=jax.ShapeDtypeStruct((NC, N, HD), jnp.float32),
    )(h3raw, deg_parts)


def _tc_epilogue(acc, h3, deg_parts, b2):
    B = 1000

    def body(acc_ref, h3_ref, dp_ref, b_ref, o_ref):
        d = dp_ref[0] + dp_ref[1] + 1.0
        dinv = lax.rsqrt(d)
        h2cat = jnp.concatenate([h3_ref[0], h3_ref[1]], axis=1)
        acccat = jnp.concatenate([acc_ref[0], acc_ref[1]], axis=1)
        o_ref[...] = jnp.maximum(dinv * (acccat + h2cat) + b_ref[...], 0.0)

    return pl.pallas_call(
        body,
        grid=(N // B,),
        in_specs=[
            pl.BlockSpec((NC, B, HD), lambda i: (0, i, 0)),
            pl.BlockSpec((NC, B, HD), lambda i: (0, i, 0)),
            pl.BlockSpec((NC, B, 1), lambda i: (0, i, 0)),
            pl.BlockSpec((1, D), lambda i: (0, 0)),
        ],
        out_specs=pl.BlockSpec((B, D), lambda i: (i, 0)),
        out_shape=jax.ShapeDtypeStruct((N, D), jnp.float32),
    )(acc, h3, deg_parts, b2)


def kernel(x, edge_index, W, b):
    src = edge_index[0].astype(jnp.int32)
    dst = edge_index[1].astype(jnp.int32)
    pad = EPAD - E
    src_p = jnp.concatenate([src, jnp.zeros((pad,), jnp.int32)])
    dst_p = jnp.concatenate([dst, jnp.full((pad,), PAD_DST, jnp.int32)])

    ones_vec = jnp.ones((CHUNK,), jnp.float32)
    zeros640 = jnp.zeros((640,), jnp.float32)
    zeros_rows = jnp.zeros((CHUNK, HD), jnp.float32)

    deg_parts = _sc_degree(dst_p, ones_vec, zeros640)
    deg_col = deg_parts[:, :, None]
    h3raw = _tc_matmul(x, W)
    h3 = _tc_scale(h3raw, deg_col)
    acc = _sc_segsum(src_p, dst_p, h3, zeros_rows)
    return _tc_epilogue(acc, h3, deg_col, b.reshape(1, D))


# speedup vs baseline: 1.0229x; 1.0229x over previous
import functools

import jax
import jax.numpy as jnp
from jax import lax
from jax.experimental import pallas as pl
from jax.experimental.pallas import tpu as pltpu
from jax.experimental.pallas import tpu_sc as plsc

N = 10000
E = 160000
D = 256
HD = 128

NC = 2
NS = 16
CHUNK = 128

EPAD = 163840
PAD_DST = 10008

ROWS = 10240

TCH = EPAD // NS
NCH = TCH // CHUNK

_mesh = plsc.VectorSubcoreMesh(core_axis_name="c", subcore_axis_name="s")


@functools.partial(
    pl.kernel,
    mesh=_mesh,
    out_type=jax.ShapeDtypeStruct((NC, ROWS), jnp.float32),
    scratch_types=[
        pltpu.VMEM((CHUNK,), jnp.int32),
        pltpu.VMEM((CHUNK,), jnp.float32),
        pltpu.VMEM((640,), jnp.float32),
        pltpu.VMEM_SHARED((ROWS,), jnp.float32),
    ],
)
def _sc_degree(dst_hbm, ones_hbm, zeros_hbm, out_hbm, idx_v, ones_v, z_v,
               acc_sh):
    c = lax.axis_index("c")
    s = lax.axis_index("s")

    pltpu.sync_copy(zeros_hbm, z_v)
    pltpu.sync_copy(z_v, acc_sh.at[pl.ds(s * 640, 640)])
    pltpu.sync_copy(ones_hbm, ones_v)
    plsc.subcore_barrier()

    base = (c * NS + s) * (EPAD // (NC * NS))

    @pl.loop(0, EPAD // (NC * NS), step=CHUNK)
    def _(k):
        pltpu.sync_copy(dst_hbm.at[pl.ds(base + k, CHUNK)], idx_v)
        pltpu.sync_copy(ones_v, acc_sh.at[idx_v], add=True)

    plsc.subcore_barrier()
    pltpu.sync_copy(acc_sh.at[pl.ds(s * 640, 640)], z_v)
    pltpu.sync_copy(z_v, out_hbm.at[c, pl.ds(s * 640, 640)])


@functools.partial(
    pl.kernel,
    mesh=_mesh,
    out_type=jax.ShapeDtypeStruct((NC, ROWS, HD), jnp.float32),
    scratch_types=[
        pltpu.VMEM((TCH,), jnp.int32),
        pltpu.VMEM((CHUNK,), jnp.int32),
        pltpu.VMEM((CHUNK,), jnp.int32),
        pltpu.VMEM((CHUNK, HD), jnp.float32),
        pltpu.VMEM((CHUNK, HD), jnp.float32),
        pltpu.VMEM_SHARED((ROWS, HD), jnp.float32),
        pltpu.SemaphoreType.DMA,
        pltpu.SemaphoreType.DMA,
        pltpu.SemaphoreType.DMA,
        pltpu.SemaphoreType.DMA,
    ],
)
def _sc_segsum(src_hbm, dst_hbm, h3_hbm, zeros_hbm, out_hbm,
               src_all, d0, d1, r0, r1, acc_sh, sem0, sem1, ssem0, ssem1):
    c = lax.axis_index("c")
    s = lax.axis_index("s")
    slots = ((d0, r0, sem0, ssem0), (d1, r1, sem1, ssem1))
    ebase = s * TCH
    h2c = h3_hbm.at[c]

    def fire(k, slot, first=False):
        dv, rv, sem, ssem = slots[slot]
        if not first:
            pltpu.make_async_copy(rv, acc_sh.at[dv], ssem).wait()
        pltpu.async_copy(h2c.at[src_all.at[pl.ds(k * CHUNK, CHUNK)]], rv, sem)
        pltpu.async_copy(dst_hbm.at[pl.ds(ebase + k * CHUNK, CHUNK)], dv, sem)

    def drain(k, slot):
        dv, rv, sem, ssem = slots[slot]
        pltpu.make_async_copy(
            h2c.at[src_all.at[pl.ds(k * CHUNK, CHUNK)]], rv, sem).wait()
        pltpu.make_async_copy(
            dst_hbm.at[pl.ds(ebase + k * CHUNK, CHUNK)], dv, sem).wait()
        pltpu.async_copy(rv, acc_sh.at[dv], ssem, add=True)

    pltpu.sync_copy(src_hbm.at[pl.ds(ebase, TCH)], src_all)

    pltpu.sync_copy(zeros_hbm, r0)
    zbase = s * 640
    for t in range(5):
        pltpu.sync_copy(r0, acc_sh.at[pl.ds(zbase + t * CHUNK, CHUNK)])

    fire(0, 0, first=True)
    fire(1, 1, first=True)
    plsc.subcore_barrier()

    @pl.loop(0, NCH, step=2)
    def _(k):
        drain(k, 0)

        @pl.when(k + 2 < NCH)
        def _():
            fire(k + 2, 0)

        drain(k + 1, 1)

        @pl.when(k + 3 < NCH)
        def _():
            fire(k + 3, 1)

    pltpu.make_async_copy(r0, acc_sh.at[d0], ssem0).wait()
    pltpu.make_async_copy(r1, acc_sh.at[d1], ssem1).wait()
    plsc.subcore_barrier()
    pltpu.sync_copy(acc_sh.at[pl.ds(zbase, 640)],
                    out_hbm.at[c, pl.ds(zbase, 640)])


def _tc_matmul(x, W):
    B = 1000

    def body(x_ref, w_ref, o_ref):
        h = jnp.dot(x_ref[...], w_ref[...],
                    preferred_element_type=jnp.float32,
                    precision=lax.Precision.HIGHEST)
        o_ref[0] = h[:, :HD]
        o_ref[1] = h[:, HD:]

    return pl.pallas_call(
        body,
        grid=(N // B,),
        in_specs=[
            pl.BlockSpec((B, D), lambda i: (i, 0)),
            pl.BlockSpec((D, D), lambda i: (0, 0)),
        ],
        out_specs=pl.BlockSpec((NC, B, HD), lambda i: (0, i, 0)),
        out_shape=jax.ShapeDtypeStruct((NC, N, HD), jnp.float32),
    )(x, W)


def _tc_scale(h3raw, deg_parts):
    B = 1000

    def body(h_ref, dp_ref, o_ref):
        d = dp_ref[0] + dp_ref[1] + 1.0
        dinv = lax.rsqrt(d)
        o_ref[...] = h_ref[...] * dinv

    return pl.pallas_call(
        body,
        grid=(N // B,),
        in_specs=[
            pl.BlockSpec((NC, B, HD), lambda i: (0, i, 0)),
            pl.BlockSpec((NC, B, 1), lambda i: (0, i, 0)),
        ],
        out_specs=pl.BlockSpec((NC, B, HD), lambda i: (0, i, 0)),
        out_shape=jax.ShapeDtypeStruct((NC, N, HD), jnp.float32),
    )(h3raw, deg_parts)


def _tc_epilogue(acc, h3, deg_parts, b2):
    B = 1000

    def body(acc_ref, h3_ref, dp_ref, b_ref, o_ref):
        d = dp_ref[0] + dp_ref[1] + 1.0
        dinv = lax.rsqrt(d)
        h2cat = jnp.concatenate([h3_ref[0], h3_ref[1]], axis=1)
        acccat = jnp.concatenate([acc_ref[0], acc_ref[1]], axis=1)
        o_ref[...] = jnp.maximum(dinv * (acccat + h2cat) + b_ref[...], 0.0)

    return pl.pallas_call(
        body,
        grid=(N // B,),
        in_specs=[
            pl.BlockSpec((NC, B, HD), lambda i: (0, i, 0)),
            pl.BlockSpec((NC, B, HD), lambda i: (0, i, 0)),
            pl.BlockSpec((NC, B, 1), lambda i: (0, i, 0)),
            pl.BlockSpec((1, D), lambda i: (0, 0)),
        ],
        out_specs=pl.BlockSpec((B, D), lambda i: (i, 0)),
        out_shape=jax.ShapeDtypeStruct((N, D), jnp.float32),
    )(acc, h3, deg_parts, b2)


def kernel(x, edge_index, W, b):
    src = edge_index[0].astype(jnp.int32)
    dst = edge_index[1].astype(jnp.int32)
    pad = EPAD - E
    src_p = jnp.concatenate([src, jnp.zeros((pad,), jnp.int32)])
    dst_p = jnp.concatenate([dst, jnp.full((pad,), PAD_DST, jnp.int32)])

    ones_vec = jnp.ones((CHUNK,), jnp.float32)
    zeros640 = jnp.zeros((640,), jnp.float32)
    zeros_rows = jnp.zeros((CHUNK, HD), jnp.float32)

    deg_parts = _sc_degree(dst_p, ones_vec, zeros640)
    deg_col = deg_parts[:, :, None]
    h3raw = _tc_matmul(x, W)
    h3 = _tc_scale(h3raw, deg_col)
    acc = _sc_segsum(src_p, dst_p, h3, zeros_rows)
    return _tc_epilogue(acc, h3, deg_col, b.reshape(1, D))


# speedup vs baseline: 1.0540x; 1.0304x over previous
import functools

import jax
import jax.numpy as jnp
from jax import lax
from jax.experimental import pallas as pl
from jax.experimental.pallas import tpu as pltpu
from jax.experimental.pallas import tpu_sc as plsc

N = 10000
E = 160000
D = 256
HD = 128

NC = 2
NS = 16
CHUNK = 128

EPAD = 163840
PAD_DST = 10008

ROWS = 10240

TCH = EPAD // NS
NCH = TCH // CHUNK

_mesh = plsc.VectorSubcoreMesh(core_axis_name="c", subcore_axis_name="s")


@functools.partial(
    pl.kernel,
    mesh=_mesh,
    out_type=jax.ShapeDtypeStruct((NC, ROWS), jnp.float32),
    scratch_types=[
        pltpu.VMEM((CHUNK,), jnp.int32),
        pltpu.VMEM((CHUNK,), jnp.int32),
        pltpu.VMEM((CHUNK,), jnp.float32),
        pltpu.VMEM((640,), jnp.float32),
        pltpu.VMEM_SHARED((ROWS,), jnp.float32),
        pltpu.SemaphoreType.DMA,
        pltpu.SemaphoreType.DMA,
    ],
)
def _sc_degree(dst_hbm, ones_hbm, zeros_hbm, out_hbm, i0, i1, ones_v, z_v,
               acc_sh, dsem0, dsem1):
    c = lax.axis_index("c")
    s = lax.axis_index("s")
    WCH = EPAD // (NC * NS)
    base = (c * NS + s) * WCH
    islots = ((i0, dsem0), (i1, dsem1))

    def ifire(k, slot):
        iv, sem = islots[slot]
        pltpu.async_copy(dst_hbm.at[pl.ds(base + k, CHUNK)], iv, sem)

    def idrain(k, slot):
        iv, sem = islots[slot]
        pltpu.make_async_copy(
            dst_hbm.at[pl.ds(base + k, CHUNK)], iv, sem).wait()
        pltpu.sync_copy(ones_v, acc_sh.at[iv], add=True)

    pltpu.sync_copy(zeros_hbm, z_v)
    pltpu.sync_copy(z_v, acc_sh.at[pl.ds(s * 640, 640)])
    pltpu.sync_copy(ones_hbm, ones_v)
    ifire(0, 0)
    ifire(CHUNK, 1)
    plsc.subcore_barrier()

    @pl.loop(0, WCH, step=2 * CHUNK)
    def _(k):
        idrain(k, 0)

        @pl.when(k + 2 * CHUNK < WCH)
        def _():
            ifire(k + 2 * CHUNK, 0)

        idrain(k + CHUNK, 1)

        @pl.when(k + 3 * CHUNK < WCH)
        def _():
            ifire(k + 3 * CHUNK, 1)

    plsc.subcore_barrier()
    pltpu.sync_copy(acc_sh.at[pl.ds(s * 640, 640)], z_v)
    pltpu.sync_copy(z_v, out_hbm.at[c, pl.ds(s * 640, 640)])


@functools.partial(
    pl.kernel,
    mesh=_mesh,
    out_type=jax.ShapeDtypeStruct((NC, ROWS, HD), jnp.float32),
    scratch_types=[
        pltpu.VMEM((TCH,), jnp.int32),
        pltpu.VMEM((CHUNK,), jnp.int32),
        pltpu.VMEM((CHUNK,), jnp.int32),
        pltpu.VMEM((CHUNK, HD), jnp.float32),
        pltpu.VMEM((CHUNK, HD), jnp.float32),
        pltpu.VMEM_SHARED((ROWS, HD), jnp.float32),
        pltpu.SemaphoreType.DMA,
        pltpu.SemaphoreType.DMA,
        pltpu.SemaphoreType.DMA,
        pltpu.SemaphoreType.DMA,
    ],
)
def _sc_segsum(src_hbm, dst_hbm, h3_hbm, zeros_hbm, out_hbm,
               src_all, d0, d1, r0, r1, acc_sh, sem0, sem1, ssem0, ssem1):
    c = lax.axis_index("c")
    s = lax.axis_index("s")
    slots = ((d0, r0, sem0, ssem0), (d1, r1, sem1, ssem1))
    ebase = s * TCH
    h2c = h3_hbm.at[c]

    def fire(k, slot, first=False):
        dv, rv, sem, ssem = slots[slot]
        if not first:
            pltpu.make_async_copy(rv, acc_sh.at[dv], ssem).wait()
        pltpu.async_copy(h2c.at[src_all.at[pl.ds(k * CHUNK, CHUNK)]], rv, sem)
        pltpu.async_copy(dst_hbm.at[pl.ds(ebase + k * CHUNK, CHUNK)], dv, sem)

    def drain(k, slot):
        dv, rv, sem, ssem = slots[slot]
        pltpu.make_async_copy(
            h2c.at[src_all.at[pl.ds(k * CHUNK, CHUNK)]], rv, sem).wait()
        pltpu.make_async_copy(
            dst_hbm.at[pl.ds(ebase + k * CHUNK, CHUNK)], dv, sem).wait()
        pltpu.async_copy(rv, acc_sh.at[dv], ssem, add=True)

    pltpu.sync_copy(src_hbm.at[pl.ds(ebase, TCH)], src_all)

    pltpu.sync_copy(zeros_hbm, r0)
    zbase = s * 640
    for t in range(5):
        pltpu.sync_copy(r0, acc_sh.at[pl.ds(zbase + t * CHUNK, CHUNK)])

    fire(0, 0, first=True)
    fire(1, 1, first=True)
    plsc.subcore_barrier()

    @pl.loop(0, NCH, step=2)
    def _(k):
        drain(k, 0)

        @pl.when(k + 2 < NCH)
        def _():
            fire(k + 2, 0)

        drain(k + 1, 1)

        @pl.when(k + 3 < NCH)
        def _():
            fire(k + 3, 1)

    pltpu.make_async_copy(r0, acc_sh.at[d0], ssem0).wait()
    pltpu.make_async_copy(r1, acc_sh.at[d1], ssem1).wait()
    plsc.subcore_barrier()
    pltpu.sync_copy(acc_sh.at[pl.ds(zbase, 640)],
                    out_hbm.at[c, pl.ds(zbase, 640)])


def _tc_matmul(x, W):
    B = 1000

    def body(x_ref, w_ref, o_ref):
        h = jnp.dot(x_ref[...], w_ref[...],
                    preferred_element_type=jnp.float32,
                    precision=lax.Precision.HIGHEST)
        o_ref[0] = h[:, :HD]
        o_ref[1] = h[:, HD:]

    return pl.pallas_call(
        body,
        grid=(N // B,),
        in_specs=[
            pl.BlockSpec((B, D), lambda i: (i, 0)),
            pl.BlockSpec((D, D), lambda i: (0, 0)),
        ],
        out_specs=pl.BlockSpec((NC, B, HD), lambda i: (0, i, 0)),
        out_shape=jax.ShapeDtypeStruct((NC, N, HD), jnp.float32),
    )(x, W)


def _tc_scale(h3raw, deg_parts):
    B = 1000

    def body(h_ref, dp_ref, o_ref):
        d = dp_ref[0] + dp_ref[1] + 1.0
        dinv = lax.rsqrt(d)
        o_ref[...] = h_ref[...] * dinv

    return pl.pallas_call(
        body,
        grid=(N // B,),
        in_specs=[
            pl.BlockSpec((NC, B, HD), lambda i: (0, i, 0)),
            pl.BlockSpec((NC, B, 1), lambda i: (0, i, 0)),
        ],
        out_specs=pl.BlockSpec((NC, B, HD), lambda i: (0, i, 0)),
        out_shape=jax.ShapeDtypeStruct((NC, N, HD), jnp.float32),
    )(h3raw, deg_parts)


def _tc_epilogue(acc, h3, deg_parts, b2):
    B = 1000

    def body(acc_ref, h3_ref, dp_ref, b_ref, o_ref):
        d = dp_ref[0] + dp_ref[1] + 1.0
        dinv = lax.rsqrt(d)
        h2cat = jnp.concatenate([h3_ref[0], h3_ref[1]], axis=1)
        acccat = jnp.concatenate([acc_ref[0], acc_ref[1]], axis=1)
        o_ref[...] = jnp.maximum(dinv * (acccat + h2cat) + b_ref[...], 0.0)

    return pl.pallas_call(
        body,
        grid=(N // B,),
        in_specs=[
            pl.BlockSpec((NC, B, HD), lambda i: (0, i, 0)),
            pl.BlockSpec((NC, B, HD), lambda i: (0, i, 0)),
            pl.BlockSpec((NC, B, 1), lambda i: (0, i, 0)),
            pl.BlockSpec((1, D), lambda i: (0, 0)),
        ],
        out_specs=pl.BlockSpec((B, D), lambda i: (i, 0)),
        out_shape=jax.ShapeDtypeStruct((N, D), jnp.float32),
    )(acc, h3, deg_parts, b2)


def kernel(x, edge_index, W, b):
    src = edge_index[0].astype(jnp.int32)
    dst = edge_index[1].astype(jnp.int32)
    pad = EPAD - E
    src_p = jnp.concatenate([src, jnp.zeros((pad,), jnp.int32)])
    dst_p = jnp.concatenate([dst, jnp.full((pad,), PAD_DST, jnp.int32)])

    ones_vec = jnp.ones((CHUNK,), jnp.float32)
    zeros640 = jnp.zeros((640,), jnp.float32)
    zeros_rows = jnp.zeros((CHUNK, HD), jnp.float32)

    deg_parts = _sc_degree(dst_p, ones_vec, zeros640)
    deg_col = deg_parts[:, :, None]
    h3raw = _tc_matmul(x, W)
    h3 = _tc_scale(h3raw, deg_col)
    acc = _sc_segsum(src_p, dst_p, h3, zeros_rows)
    return _tc_epilogue(acc, h3, deg_col, b.reshape(1, D))
